# Initial kernel scaffold; baseline (speedup 1.0000x reference)
import jax
import jax.numpy as jnp
from jax.experimental import pallas as pl


def kernel(x, edge_index, edge_attr, Wi, Wh, Wo):
    raise NotImplementedError("write your pallas kernel here")


# speedup vs baseline: 2.9071x; 2.9071x over previous
import functools

import jax
import jax.numpy as jnp
from jax import lax
from jax.experimental import pallas as pl
from jax.experimental.pallas import tpu as pltpu
from jax.experimental.pallas import tpu_sc as plsc

_NC = 2
_NS = 16
_NW = _NC * _NS

_C = 80
_G = 5
_CS = 40
_GS = 2

_BE = 2000
_BN = 1000


def _num_chunks(e_total):
    ew = e_total // _NW
    return ew // _C


def _sc_gather(table, idx3):
    n, h = table.shape
    nw, kch, c = idx3.shape
    e_total = nw * kch * c
    ew = kch * c
    mesh = plsc.VectorSubcoreMesh(core_axis_name="c", subcore_axis_name="s")

    @functools.partial(
        pl.kernel,
        out_type=jax.ShapeDtypeStruct((e_total, h), jnp.float32),
        mesh=mesh,
        scratch_types=[
            pltpu.VMEM((kch, c), jnp.int32),
            pltpu.VMEM((_G * c, h), jnp.float32),
            pltpu.SemaphoreType.DMA,
        ],
    )
    def k(table_hbm, idx_hbm, out_hbm, idx_v, rows_v, sem):
        wid = lax.axis_index("s") * _NC + lax.axis_index("c")
        base = wid * ew
        pltpu.sync_copy(idx_hbm.at[wid], idx_v)

        def body(i, carry):
            cps = []
            for j in range(_G):
                cps.append(
                    pltpu.async_copy(
                        table_hbm.at[idx_v.at[i * _G + j]],
                        rows_v.at[pl.ds(j * c, c)],
                        sem,
                    )
                )
            for cp in cps:
                cp.wait()
            pltpu.sync_copy(rows_v, out_hbm.at[pl.ds(base + i * (_G * c), _G * c)])
            return carry

        lax.fori_loop(0, kch // _G, body, 0)

    return k(table, idx3)


def _sc_scatter(rows, idx3, zeros):
    e_total, h = rows.shape
    n = zeros.shape[0]
    nw, kch, c = idx3.shape
    ew = kch * c
    rows_per_dump = 1000
    n_dumpers = n // rows_per_dump
    mesh = plsc.VectorSubcoreMesh(core_axis_name="c", subcore_axis_name="s")

    @functools.partial(
        pl.kernel,
        out_type=jax.ShapeDtypeStruct((_NC, n, h), jnp.float32),
        mesh=mesh,
        scratch_types=[
            pltpu.VMEM((kch, c), jnp.int32),
            pltpu.VMEM((_GS * c, h), jnp.float32),
            pltpu.VMEM_SHARED((n, h), jnp.float32),
            pltpu.SemaphoreType.DMA,
        ],
    )
    def k(rows_hbm, idx_hbm, zeros_hbm, out_hbm, idx_v, rows_v, acc_sh, sem):
        cid = lax.axis_index("c")
        sid = lax.axis_index("s")
        wid = sid * _NC + cid
        base = wid * ew
        pltpu.sync_copy(idx_hbm.at[wid], idx_v)

        @pl.when(sid == 0)
        def _():
            pltpu.sync_copy(zeros_hbm, acc_sh)

        plsc.subcore_barrier()

        def body(i, carry):
            pltpu.async_copy(
                rows_hbm.at[pl.ds(base + i * (_GS * c), _GS * c)], rows_v, sem
            ).wait()
            for j in range(_GS):
                pltpu.sync_copy(
                    rows_v.at[pl.ds(j * c, c)],
                    acc_sh.at[idx_v.at[i * _GS + j]],
                    add=True,
                )
            return carry

        lax.fori_loop(0, kch // _GS, body, 0)
        plsc.subcore_barrier()

        @pl.when(sid < n_dumpers)
        def _():
            pltpu.sync_copy(
                acc_sh.at[pl.ds(sid * rows_per_dump, rows_per_dump)],
                out_hbm.at[cid, pl.ds(sid * rows_per_dump, rows_per_dump)],
            )

    return k(rows, idx3, zeros)


def _mm_body(x_ref, w_ref, o_ref):
    o_ref[...] = jnp.dot(x_ref[...], w_ref[...], preferred_element_type=jnp.float32)


def _node_matmul(x, w):
    n, k = x.shape
    h = w.shape[1]
    return pl.pallas_call(
        _mm_body,
        out_shape=jax.ShapeDtypeStruct((n, h), jnp.float32),
        grid=(n // _BN,),
        in_specs=[
            pl.BlockSpec((_BN, k), lambda i: (i, 0)),
            pl.BlockSpec((k, h), lambda i: (0, 0)),
        ],
        out_specs=pl.BlockSpec((_BN, h), lambda i: (i, 0)),
    )(x, w)


def _k1_body(m0_ref, ea_ref, wie_ref, wh_ref, h0_ref, g0_ref):
    h0 = jnp.maximum(
        m0_ref[...]
        + jnp.dot(ea_ref[...], wie_ref[...], preferred_element_type=jnp.float32),
        0.0,
    )
    h0_ref[...] = h0
    g0_ref[...] = jnp.dot(h0, wh_ref[...], preferred_element_type=jnp.float32)


def _k1(m0, ea, wie, wh):
    e, h = m0.shape
    de = ea.shape[1]
    return pl.pallas_call(
        _k1_body,
        out_shape=(
            jax.ShapeDtypeStruct((e, h), jnp.float32),
            jax.ShapeDtypeStruct((e, h), jnp.float32),
        ),
        grid=(e // _BE,),
        in_specs=[
            pl.BlockSpec((_BE, h), lambda i: (i, 0)),
            pl.BlockSpec((_BE, de), lambda i: (i, 0)),
            pl.BlockSpec((de, h), lambda i: (0, 0)),
            pl.BlockSpec((h, h), lambda i: (0, 0)),
        ],
        out_specs=(
            pl.BlockSpec((_BE, h), lambda i: (i, 0)),
            pl.BlockSpec((_BE, h), lambda i: (i, 0)),
        ),
    )(m0, ea, wie, wh)


def _rev_swap(g):
    gm1 = pltpu.roll(g, g.shape[0] - 1, 0)
    gp1 = pltpu.roll(g, 1, 0)
    row = lax.broadcasted_iota(jnp.int32, g.shape, 0)
    return jnp.where((row & 1) == 0, gm1, gp1)


def _k2_body(h0_ref, m_ref, g_ref, wh_ref, o_ref):
    hn = jnp.maximum(h0_ref[...] + m_ref[...] - _rev_swap(g_ref[...]), 0.0)
    o_ref[...] = jnp.dot(hn, wh_ref[...], preferred_element_type=jnp.float32)


def _k2(h0, m, g, wh):
    e, h = h0.shape
    return pl.pallas_call(
        _k2_body,
        out_shape=jax.ShapeDtypeStruct((e, h), jnp.float32),
        grid=(e // _BE,),
        in_specs=[
            pl.BlockSpec((_BE, h), lambda i: (i, 0)),
            pl.BlockSpec((_BE, h), lambda i: (i, 0)),
            pl.BlockSpec((_BE, h), lambda i: (i, 0)),
            pl.BlockSpec((h, h), lambda i: (0, 0)),
        ],
        out_specs=pl.BlockSpec((_BE, h), lambda i: (i, 0)),
    )(h0, m, g, wh)


def _k3_body(h0_ref, m_ref, g_ref, o_ref):
    o_ref[...] = jnp.maximum(h0_ref[...] + m_ref[...] - _rev_swap(g_ref[...]), 0.0)


def _k3(h0, m, g):
    e, h = h0.shape
    return pl.pallas_call(
        _k3_body,
        out_shape=jax.ShapeDtypeStruct((e, h), jnp.float32),
        grid=(e // _BE,),
        in_specs=[
            pl.BlockSpec((_BE, h), lambda i: (i, 0)),
            pl.BlockSpec((_BE, h), lambda i: (i, 0)),
            pl.BlockSpec((_BE, h), lambda i: (i, 0)),
        ],
        out_specs=pl.BlockSpec((_BE, h), lambda i: (i, 0)),
    )(h0, m, g)


def _add_body(p_ref, a_ref, o_ref):
    o_ref[...] = p_ref[0] + a_ref[0]


def _sum_parts(p):
    nc, n, h = p.shape
    return pl.pallas_call(
        _add_body,
        out_shape=jax.ShapeDtypeStruct((n, h), jnp.float32),
        grid=(n // _BN,),
        in_specs=[
            pl.BlockSpec((1, _BN, h), lambda i: (0, i, 0)),
            pl.BlockSpec((1, _BN, h), lambda i: (1, i, 0)),
        ],
        out_specs=pl.BlockSpec((_BN, h), lambda i: (i, 0)),
    )(p, p)


def _final_body(x_ref, p_ref, a_ref, wx_ref, wm_ref, o_ref):
    mv = p_ref[0] + a_ref[0]
    o_ref[...] = jnp.maximum(
        jnp.dot(x_ref[...], wx_ref[...], preferred_element_type=jnp.float32)
        + jnp.dot(mv, wm_ref[...], preferred_element_type=jnp.float32),
        0.0,
    )


def _final(x, p, wx, wm):
    n, dn = x.shape
    h = wx.shape[1]
    return pl.pallas_call(
        _final_body,
        out_shape=jax.ShapeDtypeStruct((n, h), jnp.float32),
        grid=(n // _BN,),
        in_specs=[
            pl.BlockSpec((_BN, dn), lambda i: (i, 0)),
            pl.BlockSpec((1, _BN, h), lambda i: (0, i, 0)),
            pl.BlockSpec((1, _BN, h), lambda i: (1, i, 0)),
            pl.BlockSpec((dn, h), lambda i: (0, 0)),
            pl.BlockSpec((h, h), lambda i: (0, 0)),
        ],
        out_specs=pl.BlockSpec((_BN, h), lambda i: (i, 0)),
    )(x, p, p, wx, wm)


def kernel(x, edge_index, edge_attr, Wi, Wh, Wo):
    n, dn = x.shape
    e_total = edge_index.shape[1]

    src3 = edge_index[0].astype(jnp.int32).reshape(_NW, e_total // (_NW * _C), _C)
    dst3 = edge_index[1].astype(jnp.int32).reshape(_NW, e_total // (_NW * _CS), _CS)
    wi_x, wi_e = Wi[:dn], Wi[dn:]
    wo_x, wo_m = Wo[:dn], Wo[dn:]
    zeros = jnp.zeros((n, Wh.shape[0]), jnp.float32)

    xw = _node_matmul(x, wi_x)
    m0 = _sc_gather(xw, src3)
    h0, g0 = _k1(m0, edge_attr, wi_e, Wh)
    p0 = _sc_scatter(g0, dst3, zeros)
    s0 = _sum_parts(p0)
    m1 = _sc_gather(s0, src3)
    g1 = _k2(h0, m1, g0, Wh)
    p1 = _sc_scatter(g1, dst3, zeros)
    s1 = _sum_parts(p1)
    m2 = _sc_gather(s1, src3)
    h2 = _k3(h0, m2, g1)
    pm = _sc_scatter(h2, dst3, zeros)
    return _final(x, pm, wo_x, wo_m)


# speedup vs baseline: 3.3093x; 1.1383x over previous
import functools

import jax
import jax.numpy as jnp
from jax import lax
from jax.experimental import pallas as pl
from jax.experimental.pallas import tpu as pltpu
from jax.experimental.pallas import tpu_sc as plsc

_NC = 2
_NS = 16
_NW = _NC * _NS

_C = 80
_CS = 40

_BE = 2000
_BN = 1000


def _num_chunks(e_total):
    ew = e_total // _NW
    return ew // _C


def _sc_gather(table, idx3):
    n, h = table.shape
    nw, kch, c = idx3.shape
    e_total = nw * kch * c
    ew = kch * c
    mesh = plsc.VectorSubcoreMesh(core_axis_name="c", subcore_axis_name="s")

    @functools.partial(
        pl.kernel,
        out_type=jax.ShapeDtypeStruct((e_total, h), jnp.float32),
        mesh=mesh,
        scratch_types=[
            pltpu.VMEM((kch, c), jnp.int32),
            pltpu.VMEM((c, h), jnp.float32),
            pltpu.VMEM((c, h), jnp.float32),
            pltpu.VMEM_SHARED((n, h), jnp.float32),
            pltpu.SemaphoreType.DMA,
            pltpu.SemaphoreType.DMA,
        ],
    )
    def k(table_hbm, idx_hbm, out_hbm, idx_v, rows_a, rows_b, table_sh, gsem, ssem):
        cid = lax.axis_index("c")
        sid = lax.axis_index("s")
        wid = sid * _NC + cid
        base = wid * ew
        pltpu.sync_copy(idx_hbm.at[wid], idx_v)

        @pl.when(sid == 0)
        def _():
            pltpu.sync_copy(table_hbm, table_sh)

        plsc.subcore_barrier()

        def body(i, carry):
            cp_a = pltpu.async_copy(table_sh.at[idx_v.at[2 * i]], rows_a, gsem)
            cp_b = pltpu.async_copy(table_sh.at[idx_v.at[2 * i + 1]], rows_b, gsem)
            cp_a.wait()
            st_a = pltpu.async_copy(
                rows_a, out_hbm.at[pl.ds(base + 2 * i * c, c)], ssem
            )
            cp_b.wait()
            st_b = pltpu.async_copy(
                rows_b, out_hbm.at[pl.ds(base + (2 * i + 1) * c, c)], ssem
            )
            st_a.wait()
            st_b.wait()
            return carry

        lax.fori_loop(0, kch // 2, body, 0)
        if kch % 2:
            j = kch - 1
            pltpu.async_copy(table_sh.at[idx_v.at[j]], rows_a, gsem).wait()
            pltpu.async_copy(rows_a, out_hbm.at[pl.ds(base + j * c, c)], ssem).wait()

    return k(table, idx3)


def _sc_scatter(rows, idx3, zeros):
    e_total, h = rows.shape
    n = zeros.shape[0]
    nw, kch, c = idx3.shape
    ew = kch * c
    rows_per_dump = 1000
    n_dumpers = n // rows_per_dump
    mesh = plsc.VectorSubcoreMesh(core_axis_name="c", subcore_axis_name="s")

    @functools.partial(
        pl.kernel,
        out_type=jax.ShapeDtypeStruct((_NC, n, h), jnp.float32),
        mesh=mesh,
        scratch_types=[
            pltpu.VMEM((kch, c), jnp.int32),
            pltpu.VMEM((c, h), jnp.float32),
            pltpu.VMEM((c, h), jnp.float32),
            pltpu.VMEM_SHARED((n, h), jnp.float32),
            pltpu.SemaphoreType.DMA,
            pltpu.SemaphoreType.DMA,
        ],
    )
    def k(rows_hbm, idx_hbm, zeros_hbm, out_hbm, idx_v, rows_a, rows_b, acc_sh,
          lsem, asem):
        cid = lax.axis_index("c")
        sid = lax.axis_index("s")
        wid = sid * _NC + cid
        base = wid * ew
        pltpu.sync_copy(idx_hbm.at[wid], idx_v)

        @pl.when(sid == 0)
        def _():
            pltpu.sync_copy(zeros_hbm, acc_sh)

        plsc.subcore_barrier()

        def body(i, carry):
            cp_a = pltpu.async_copy(
                rows_hbm.at[pl.ds(base + 2 * i * c, c)], rows_a, lsem
            )
            cp_b = pltpu.async_copy(
                rows_hbm.at[pl.ds(base + (2 * i + 1) * c, c)], rows_b, lsem
            )
            cp_a.wait()
            sc_a = pltpu.async_copy(
                rows_a, acc_sh.at[idx_v.at[2 * i]], asem, add=True
            )
            cp_b.wait()
            sc_b = pltpu.async_copy(
                rows_b, acc_sh.at[idx_v.at[2 * i + 1]], asem, add=True
            )
            sc_a.wait()
            sc_b.wait()
            return carry

        lax.fori_loop(0, kch // 2, body, 0)
        if kch % 2:
            j = kch - 1
            pltpu.async_copy(rows_hbm.at[pl.ds(base + j * c, c)], rows_a, lsem).wait()
            pltpu.async_copy(rows_a, acc_sh.at[idx_v.at[j]], asem, add=True).wait()
        plsc.subcore_barrier()

        @pl.when(sid < n_dumpers)
        def _():
            pltpu.sync_copy(
                acc_sh.at[pl.ds(sid * rows_per_dump, rows_per_dump)],
                out_hbm.at[cid, pl.ds(sid * rows_per_dump, rows_per_dump)],
            )

    return k(rows, idx3, zeros)


def _mm_body(x_ref, w_ref, o_ref):
    o_ref[...] = jnp.dot(x_ref[...], w_ref[...], preferred_element_type=jnp.float32)


def _node_matmul(x, w):
    n, k = x.shape
    h = w.shape[1]
    return pl.pallas_call(
        _mm_body,
        out_shape=jax.ShapeDtypeStruct((n, h), jnp.float32),
        grid=(n // _BN,),
        in_specs=[
            pl.BlockSpec((_BN, k), lambda i: (i, 0)),
            pl.BlockSpec((k, h), lambda i: (0, 0)),
        ],
        out_specs=pl.BlockSpec((_BN, h), lambda i: (i, 0)),
    )(x, w)


def _k1_body(m0_ref, ea_ref, wie_ref, wh_ref, h0_ref, g0_ref):
    h0 = jnp.maximum(
        m0_ref[...]
        + jnp.dot(ea_ref[...], wie_ref[...], preferred_element_type=jnp.float32),
        0.0,
    )
    h0_ref[...] = h0
    g0_ref[...] = jnp.dot(h0, wh_ref[...], preferred_element_type=jnp.float32)


def _k1(m0, ea, wie, wh):
    e, h = m0.shape
    de = ea.shape[1]
    return pl.pallas_call(
        _k1_body,
        out_shape=(
            jax.ShapeDtypeStruct((e, h), jnp.float32),
            jax.ShapeDtypeStruct((e, h), jnp.float32),
        ),
        grid=(e // _BE,),
        in_specs=[
            pl.BlockSpec((_BE, h), lambda i: (i, 0)),
            pl.BlockSpec((_BE, de), lambda i: (i, 0)),
            pl.BlockSpec((de, h), lambda i: (0, 0)),
            pl.BlockSpec((h, h), lambda i: (0, 0)),
        ],
        out_specs=(
            pl.BlockSpec((_BE, h), lambda i: (i, 0)),
            pl.BlockSpec((_BE, h), lambda i: (i, 0)),
        ),
    )(m0, ea, wie, wh)


def _rev_swap(g):
    gm1 = pltpu.roll(g, g.shape[0] - 1, 0)
    gp1 = pltpu.roll(g, 1, 0)
    row = lax.broadcasted_iota(jnp.int32, g.shape, 0)
    return jnp.where((row & 1) == 0, gm1, gp1)


def _k2_body(h0_ref, m_ref, g_ref, wh_ref, o_ref):
    hn = jnp.maximum(h0_ref[...] + m_ref[...] - _rev_swap(g_ref[...]), 0.0)
    o_ref[...] = jnp.dot(hn, wh_ref[...], preferred_element_type=jnp.float32)


def _k2(h0, m, g, wh):
    e, h = h0.shape
    return pl.pallas_call(
        _k2_body,
        out_shape=jax.ShapeDtypeStruct((e, h), jnp.float32),
        grid=(e // _BE,),
        in_specs=[
            pl.BlockSpec((_BE, h), lambda i: (i, 0)),
            pl.BlockSpec((_BE, h), lambda i: (i, 0)),
            pl.BlockSpec((_BE, h), lambda i: (i, 0)),
            pl.BlockSpec((h, h), lambda i: (0, 0)),
        ],
        out_specs=pl.BlockSpec((_BE, h), lambda i: (i, 0)),
    )(h0, m, g, wh)


def _k3_body(h0_ref, m_ref, g_ref, o_ref):
    o_ref[...] = jnp.maximum(h0_ref[...] + m_ref[...] - _rev_swap(g_ref[...]), 0.0)


def _k3(h0, m, g):
    e, h = h0.shape
    return pl.pallas_call(
        _k3_body,
        out_shape=jax.ShapeDtypeStruct((e, h), jnp.float32),
        grid=(e // _BE,),
        in_specs=[
            pl.BlockSpec((_BE, h), lambda i: (i, 0)),
            pl.BlockSpec((_BE, h), lambda i: (i, 0)),
            pl.BlockSpec((_BE, h), lambda i: (i, 0)),
        ],
        out_specs=pl.BlockSpec((_BE, h), lambda i: (i, 0)),
    )(h0, m, g)


def _add_body(p_ref, a_ref, o_ref):
    o_ref[...] = p_ref[0] + a_ref[0]


def _sum_parts(p):
    nc, n, h = p.shape
    return pl.pallas_call(
        _add_body,
        out_shape=jax.ShapeDtypeStruct((n, h), jnp.float32),
        grid=(n // _BN,),
        in_specs=[
            pl.BlockSpec((1, _BN, h), lambda i: (0, i, 0)),
            pl.BlockSpec((1, _BN, h), lambda i: (1, i, 0)),
        ],
        out_specs=pl.BlockSpec((_BN, h), lambda i: (i, 0)),
    )(p, p)


def _final_body(x_ref, p_ref, a_ref, wx_ref, wm_ref, o_ref):
    mv = p_ref[0] + a_ref[0]
    o_ref[...] = jnp.maximum(
        jnp.dot(x_ref[...], wx_ref[...], preferred_element_type=jnp.float32)
        + jnp.dot(mv, wm_ref[...], preferred_element_type=jnp.float32),
        0.0,
    )


def _final(x, p, wx, wm):
    n, dn = x.shape
    h = wx.shape[1]
    return pl.pallas_call(
        _final_body,
        out_shape=jax.ShapeDtypeStruct((n, h), jnp.float32),
        grid=(n // _BN,),
        in_specs=[
            pl.BlockSpec((_BN, dn), lambda i: (i, 0)),
            pl.BlockSpec((1, _BN, h), lambda i: (0, i, 0)),
            pl.BlockSpec((1, _BN, h), lambda i: (1, i, 0)),
            pl.BlockSpec((dn, h), lambda i: (0, 0)),
            pl.BlockSpec((h, h), lambda i: (0, 0)),
        ],
        out_specs=pl.BlockSpec((_BN, h), lambda i: (i, 0)),
    )(x, p, p, wx, wm)


def kernel(x, edge_index, edge_attr, Wi, Wh, Wo):
    n, dn = x.shape
    e_total = edge_index.shape[1]

    src3 = edge_index[0].astype(jnp.int32).reshape(_NW, e_total // (_NW * _C), _C)
    dst3 = edge_index[1].astype(jnp.int32).reshape(_NW, e_total // (_NW * _CS), _CS)
    wi_x, wi_e = Wi[:dn], Wi[dn:]
    wo_x, wo_m = Wo[:dn], Wo[dn:]
    zeros = jnp.zeros((n, Wh.shape[0]), jnp.float32)

    xw = _node_matmul(x, wi_x)
    m0 = _sc_gather(xw, src3)
    h0, g0 = _k1(m0, edge_attr, wi_e, Wh)
    p0 = _sc_scatter(g0, dst3, zeros)
    s0 = _sum_parts(p0)
    m1 = _sc_gather(s0, src3)
    g1 = _k2(h0, m1, g0, Wh)
    p1 = _sc_scatter(g1, dst3, zeros)
    s1 = _sum_parts(p1)
    m2 = _sc_gather(s1, src3)
    h2 = _k3(h0, m2, g1)
    pm = _sc_scatter(h2, dst3, zeros)
    return _final(x, pm, wo_x, wo_m)
